# baseline (speedup 1.0000x reference)
import functools

import jax
import jax.numpy as jnp
from jax import lax
from jax.experimental import pallas as pl
from jax.experimental.pallas import tpu as pltpu
from jax.experimental.pallas import tpu_sc as plsc

_N = 10000
_E = 160000
_DIM = 32
_B = 64

_NC = 2
_NS = 16
_NW = _NC * _NS
_IL = 128
_EP = 163840
_NIR = _EP // _IL
_RPT = _NIR // _NW
_NP = 10112
_RPS = _NP // _NS

_DW = 128
_EB = 512
_EB1 = 2048

def _leaky(v):
    return jnp.where(v >= 0, v, 0.01 * v)


@functools.cache
def _sc_kernels():
    mesh = plsc.VectorSubcoreMesh(core_axis_name="c", subcore_axis_name="s",
                                  num_cores=_NC, num_subcores=_NS)

    @functools.partial(
        pl.kernel,
        out_type=jax.ShapeDtypeStruct((_EP, _DW), jnp.float32),
        mesh=mesh,
        scratch_types=[
            pltpu.VMEM((_RPT, _IL), jnp.int32),
            pltpu.VMEM((_IL, _DW), jnp.float32),
            pltpu.SemaphoreType.DMA,
        ],
    )
    def sc_gather(table, idx, out, idxbuf, rowbuf, sem):
        wid = lax.axis_index("s") * _NC + lax.axis_index("c")
        base = wid * _RPT
        pltpu.sync_copy(idx.at[pl.ds(base, _RPT)], idxbuf)

        def body(j, carry):
            pltpu.async_copy(table.at[idxbuf.at[j]], rowbuf, sem).wait()
            pltpu.sync_copy(rowbuf, out.at[pl.ds((base + j) * _IL, _IL)])
            return carry

        lax.fori_loop(0, _RPT, body, 0)

    @functools.partial(
        pl.kernel,
        out_type=jax.ShapeDtypeStruct((_NC, _NP, _DW), jnp.float32),
        mesh=mesh,
        scratch_types=[
            pltpu.VMEM((_RPT, _IL), jnp.int32),
            pltpu.VMEM((_IL, _DW), jnp.float32),
            pltpu.VMEM_SHARED((_NP, _DW), jnp.float32),
        ],
    )
    def sc_scatter(msgs, idx, zeros, out, idxbuf, msgbuf, acc):
        cid = lax.axis_index("c")
        sid = lax.axis_index("s")
        wid = sid * _NC + cid
        pltpu.sync_copy(zeros.at[pl.ds(sid * _RPS, _RPS)],
                        acc.at[pl.ds(sid * _RPS, _RPS)])
        plsc.subcore_barrier()
        base = wid * _RPT
        pltpu.sync_copy(idx.at[pl.ds(base, _RPT)], idxbuf)

        def body(j, carry):
            pltpu.sync_copy(msgs.at[pl.ds((base + j) * _IL, _IL)], msgbuf)
            pltpu.sync_copy(msgbuf, acc.at[idxbuf.at[j]], add=True)
            return carry

        lax.fori_loop(0, _RPT, body, 0)
        plsc.subcore_barrier()
        pltpu.sync_copy(acc.at[pl.ds(sid * _RPS, _RPS)],
                        out.at[cid].at[pl.ds(sid * _RPS, _RPS)])

    return sc_gather, sc_scatter


def _pad_dw(v):
    return jnp.concatenate(
        [v, jnp.zeros((v.shape[0], _DW - _DIM), jnp.float32)], axis=1)


def _prep_body(x_ref, w_ref, b_ref, dp_ref, out_ref, inv_ref):
    o = _leaky(
        jnp.dot(x_ref[...], w_ref[...], preferred_element_type=jnp.float32)
        + b_ref[...])
    out_ref[...] = _pad_dw(o)
    deg = dp_ref[0, :, 0:_DIM] + dp_ref[1, :, 0:_DIM]
    inv_ref[...] = 1.0 / jnp.maximum(deg, 1.0)


def _h1_body(ea_ref, w_ref, b_ref, h1_ref):
    h1_ref[...] = _leaky(
        jnp.dot(ea_ref[...], w_ref[...], preferred_element_type=jnp.float32)
        + b_ref[...])


def _msg_body(h1_ref, g_ref, w3_ref, b2_ref, msg_ref):
    h1 = h1_ref[...]
    g = g_ref[:, 0:_DIM]
    hrep = jnp.reshape(
        jnp.broadcast_to(h1[:, :, None], (_EB, _DIM, _DIM)), (_EB, _DIM * _DIM))
    grep = jnp.concatenate([g] * _DIM, axis=1)
    p = hrep * grep
    msg = (jnp.dot(p, w3_ref[...], preferred_element_type=jnp.float32)
           + jnp.dot(g, b2_ref[...], preferred_element_type=jnp.float32))
    msg_ref[...] = _pad_dw(msg)


def _node_body(dp, inv, out, h, rwt, cb, wirt, wizt, wint, whrt, whzt, whnt,
               bir, biz, binn, bhr, bhz, bhn, h_new):
    agg = (dp[0, :, 0:_DIM] + dp[1, :, 0:_DIM]) * inv[...]
    o = out[:, 0:_DIM]
    hh = h[:, 0:_DIM]
    m = _leaky(agg + jnp.dot(o, rwt[...], preferred_element_type=jnp.float32)
               + cb[...])
    gh_r = jnp.dot(hh, whrt[...], preferred_element_type=jnp.float32) + bhr[...]
    gh_z = jnp.dot(hh, whzt[...], preferred_element_type=jnp.float32) + bhz[...]
    gh_n = jnp.dot(hh, whnt[...], preferred_element_type=jnp.float32) + bhn[...]
    r = jax.nn.sigmoid(
        jnp.dot(m, wirt[...], preferred_element_type=jnp.float32) + bir[...]
        + gh_r)
    z = jax.nn.sigmoid(
        jnp.dot(m, wizt[...], preferred_element_type=jnp.float32) + biz[...]
        + gh_z)
    n = jnp.tanh(
        jnp.dot(m, wint[...], preferred_element_type=jnp.float32) + binn[...]
        + r * gh_n)
    h_new[...] = _pad_dw((1.0 - z) * n + z * hh)


def _s2s_body(out_ref, bcol_ref, brow_ref, gi_ref, gf_ref, gg_ref, go_ref,
              wq_ref, wr_ref, ob_ref, res_ref):
    o = out_ref[0:_N, 0:_DIM]
    bcol = bcol_ref[...]
    brow = brow_ref[...]
    i_g = jax.nn.sigmoid(gi_ref[...])
    f_g = jax.nn.sigmoid(gf_ref[...])
    g_g = jnp.tanh(gg_ref[...])
    o_g = jax.nn.sigmoid(go_ref[...])
    del f_g
    q = o_g * jnp.tanh(i_g * g_g)
    e = jnp.sum(o * q, axis=1, keepdims=True)
    iota_row = lax.broadcasted_iota(jnp.int32, (1, _B), 1)
    ohb = bcol == iota_row
    ohf = ohb.astype(jnp.float32)
    neg = jnp.float32(-jnp.inf)
    emax = jnp.max(jnp.where(ohb, e, neg), axis=0, keepdims=True)
    emax = jnp.where(jnp.isfinite(emax), emax, 0.0)
    emaxn = jnp.sum(ohf * emax, axis=1, keepdims=True)
    a = jnp.exp(e - emaxn)
    denom = jnp.sum(ohf * a, axis=0, keepdims=True)
    denomn = jnp.sum(ohf * jnp.maximum(denom, 1e-16), axis=1, keepdims=True)
    an = a / denomn
    iota_col = lax.broadcasted_iota(jnp.int32, (_B, 1), 0)
    oht = (brow == iota_col).astype(jnp.float32)
    rread = jnp.dot(oht, o * an, preferred_element_type=jnp.float32)
    res_ref[...] = (
        jnp.dot(jnp.broadcast_to(q, (_B, _DIM)), wq_ref[...],
                preferred_element_type=jnp.float32)
        + jnp.dot(rread, wr_ref[...], preferred_element_type=jnp.float32)
        + ob_ref[...])


def _full(shape):
    return pl.BlockSpec(shape, lambda *_: tuple(0 for _ in shape))


def kernel(x, edge_index, edge_attr, batch, lin0_W, lin0_b, net1_W, net1_b,
           net2_W, net2_b, root_W, conv_b, gru_Wih, gru_Whh, gru_bih, gru_bhh,
           lstm_Wih, lstm_Whh, lstm_bih, lstm_bhh, out_W, out_b):
    f32 = jnp.float32
    src = edge_index[0]
    dst = edge_index[1]
    srcp = jnp.pad(src, (0, _EP - _E)).reshape(_NIR, _IL)
    dstp = jnp.pad(dst, (0, _EP - _E), constant_values=_N).reshape(_NIR, _IL)
    eap = jnp.pad(edge_attr, ((0, _EP - _E), (0, 4)))
    xp = jnp.pad(x, ((0, _NP - _N), (0, 2)))

    lin0_Wt = jnp.pad(lin0_W.T, ((0, 2), (0, 0)))
    lin0_b2 = lin0_b.reshape(1, _DIM)
    net1_Wt = jnp.pad(net1_W.T, ((0, 4), (0, 0)))
    net1_b2 = net1_b.reshape(1, _DIM)
    w3mat = net2_W.reshape(_DIM, _DIM, _DIM).transpose(2, 0, 1).reshape(
        _DIM * _DIM, _DIM)
    b2mat = net2_b.reshape(_DIM, _DIM)
    root_Wt = root_W.T
    conv_b2 = conv_b.reshape(1, _DIM)
    wirt = gru_Wih[0:_DIM].T
    wizt = gru_Wih[_DIM:2 * _DIM].T
    wint = gru_Wih[2 * _DIM:].T
    whrt = gru_Whh[0:_DIM].T
    whzt = gru_Whh[_DIM:2 * _DIM].T
    whnt = gru_Whh[2 * _DIM:].T
    bir = gru_bih[0:_DIM].reshape(1, _DIM)
    biz = gru_bih[_DIM:2 * _DIM].reshape(1, _DIM)
    binn = gru_bih[2 * _DIM:].reshape(1, _DIM)
    bhr = gru_bhh[0:_DIM].reshape(1, _DIM)
    bhz = gru_bhh[_DIM:2 * _DIM].reshape(1, _DIM)
    bhn = gru_bhh[2 * _DIM:].reshape(1, _DIM)
    lstm_b = (lstm_bih + lstm_bhh)
    gi_b = lstm_b[0:_DIM].reshape(1, _DIM)
    gf_b = lstm_b[_DIM:2 * _DIM].reshape(1, _DIM)
    gg_b = lstm_b[2 * _DIM:3 * _DIM].reshape(1, _DIM)
    go_b = lstm_b[3 * _DIM:].reshape(1, _DIM)
    out_Wt = out_W.T
    wq = out_Wt[0:_DIM]
    wr = out_Wt[_DIM:]
    ob = out_b.reshape(1, 2)
    bcol = batch.reshape(_N, 1)
    brow = batch.reshape(1, _N)

    zeros_nd = jnp.zeros((_NP, _DW), f32)
    ones_e = jnp.ones((_EP, _DW), f32)

    sc_gather, sc_scatter = _sc_kernels()
    degp = sc_scatter(ones_e, dstp, zeros_nd)

    out0, invdeg = pl.pallas_call(
        _prep_body,
        in_specs=[_full((_NP, 16)), _full((16, _DIM)), _full((1, _DIM)),
                  _full((_NC, _NP, _DW))],
        out_specs=[_full((_NP, _DW)), _full((_NP, _DIM))],
        out_shape=[jax.ShapeDtypeStruct((_NP, _DW), f32),
                   jax.ShapeDtypeStruct((_NP, _DIM), f32)],
    )(xp, lin0_Wt, lin0_b2, degp)

    h1 = pl.pallas_call(
        _h1_body,
        grid=(_EP // _EB1,),
        in_specs=[pl.BlockSpec((_EB1, 8), lambda i: (i, 0)),
                  pl.BlockSpec((8, _DIM), lambda i: (0, 0)),
                  pl.BlockSpec((1, _DIM), lambda i: (0, 0))],
        out_specs=pl.BlockSpec((_EB1, _DIM), lambda i: (i, 0)),
        out_shape=jax.ShapeDtypeStruct((_EP, _DIM), f32),
    )(eap, net1_Wt, net1_b2)

    msg_call = pl.pallas_call(
        _msg_body,
        grid=(_EP // _EB,),
        in_specs=[pl.BlockSpec((_EB, _DIM), lambda i: (i, 0)),
                  pl.BlockSpec((_EB, _DW), lambda i: (i, 0)),
                  pl.BlockSpec((_DIM * _DIM, _DIM), lambda i: (0, 0)),
                  pl.BlockSpec((_DIM, _DIM), lambda i: (0, 0))],
        out_specs=pl.BlockSpec((_EB, _DW), lambda i: (i, 0)),
        out_shape=jax.ShapeDtypeStruct((_EP, _DW), f32),
    )

    node_call = pl.pallas_call(
        _node_body,
        in_specs=[_full((_NC, _NP, _DW)), _full((_NP, _DIM)),
                  _full((_NP, _DW)), _full((_NP, _DW))]
        + [_full((_DIM, _DIM))] + [_full((1, _DIM))]
        + [_full((_DIM, _DIM))] * 6 + [_full((1, _DIM))] * 6,
        out_specs=_full((_NP, _DW)),
        out_shape=jax.ShapeDtypeStruct((_NP, _DW), f32),
    )

    h = out0
    for _ in range(6):
        gath = sc_gather(h, srcp)
        msg = msg_call(h1, gath, w3mat, b2mat)
        parts = sc_scatter(msg, dstp, zeros_nd)
        h = node_call(parts, invdeg, h, h, root_Wt, conv_b2,
                      wirt, wizt, wint, whrt, whzt, whnt,
                      bir, biz, binn, bhr, bhz, bhn)

    res = pl.pallas_call(
        _s2s_body,
        in_specs=[_full((_NP, _DW)), _full((_N, 1)), _full((1, _N))]
        + [_full((1, _DIM))] * 4 + [_full((_DIM, 2))] * 2 + [_full((1, 2))],
        out_specs=_full((_B, 2)),
        out_shape=jax.ShapeDtypeStruct((_B, 2), f32),
    )(h, bcol, brow, gi_b, gf_b, gg_b, go_b, wq, wr, ob)
    return res


# speedup vs baseline: 2.7915x; 2.7915x over previous
import functools

import jax
import jax.numpy as jnp
from jax import lax
from jax.experimental import pallas as pl
from jax.experimental.pallas import tpu as pltpu
from jax.experimental.pallas import tpu_sc as plsc

_N = 10000
_E = 160000
_DIM = 32
_B = 64

_NC = 2
_NS = 16
_NW = _NC * _NS
_IL = 128
_EP = 163840
_NIR = _EP // _IL
_RPT = _NIR // _NW
_NP = 10112
_RPS = _NP // _NS

_DW = 128
_EB = 512
_EB1 = 2048

def _leaky(v):
    return jnp.where(v >= 0, v, 0.01 * v)


@functools.cache
def _sc_kernels():
    mesh = plsc.VectorSubcoreMesh(core_axis_name="c", subcore_axis_name="s",
                                  num_cores=_NC, num_subcores=_NS)

    nbuf = 4
    ngrp = _RPT // nbuf

    @functools.partial(
        pl.kernel,
        out_type=jax.ShapeDtypeStruct((_EP, _DW), jnp.float32),
        mesh=mesh,
        scratch_types=[pltpu.VMEM((_RPT, _IL), jnp.int32)]
        + [pltpu.VMEM((_IL, _DW), jnp.float32)] * nbuf
        + [pltpu.SemaphoreType.DMA] * (2 * nbuf),
    )
    def sc_gather(table, idx, out, idxbuf, *bufs_sems):
        rbs = bufs_sems[:nbuf]
        gss = bufs_sems[nbuf:2 * nbuf]
        sss = bufs_sems[2 * nbuf:]
        wid = lax.axis_index("s") * _NC + lax.axis_index("c")
        base = wid * _RPT
        pltpu.sync_copy(idx.at[pl.ds(base, _RPT)], idxbuf)
        for b in range(nbuf):
            pltpu.async_copy(table.at[idxbuf.at[b]], rbs[b], gss[b])

        def outer(g, carry):
            for b in range(nbuf):
                j = g * nbuf + b
                pltpu.make_async_copy(table.at[idxbuf.at[b]], rbs[b],
                                      gss[b]).wait()
                pltpu.async_copy(rbs[b], out.at[pl.ds((base + j) * _IL, _IL)],
                                 sss[b])

            @pl.when(g < ngrp - 1)
            def _():
                for b in range(nbuf):
                    j = g * nbuf + b
                    pltpu.make_async_copy(
                        rbs[b], out.at[pl.ds((base + j) * _IL, _IL)],
                        sss[b]).wait()
                    pltpu.async_copy(table.at[idxbuf.at[(g + 1) * nbuf + b]],
                                     rbs[b], gss[b])

            return carry

        lax.fori_loop(0, ngrp, outer, 0)
        for b in range(nbuf):
            j = (ngrp - 1) * nbuf + b
            pltpu.make_async_copy(rbs[b], out.at[pl.ds((base + j) * _IL, _IL)],
                                  sss[b]).wait()

    chk = 1
    nmb = 2
    nchk = _RPT // chk
    ngrp2 = nchk // nmb

    @functools.partial(
        pl.kernel,
        out_type=jax.ShapeDtypeStruct((_NC, _NP, _DW), jnp.float32),
        mesh=mesh,
        scratch_types=[pltpu.VMEM((_RPT, _IL), jnp.int32)]
        + [pltpu.VMEM((chk * _IL, _DW), jnp.float32)] * nmb
        + [pltpu.SemaphoreType.DMA] * nmb
        + [pltpu.VMEM_SHARED((_NP, _DW), jnp.float32)],
    )
    def sc_scatter(msgs, idx, zeros, out, idxbuf, *bufs):
        mbs = bufs[:nmb]
        lss = bufs[nmb:2 * nmb]
        acc = bufs[2 * nmb]
        cid = lax.axis_index("c")
        sid = lax.axis_index("s")
        wid = sid * _NC + cid
        pltpu.sync_copy(zeros.at[pl.ds(sid * _RPS, _RPS)],
                        acc.at[pl.ds(sid * _RPS, _RPS)])
        plsc.subcore_barrier()
        base = wid * _RPT
        pltpu.sync_copy(idx.at[pl.ds(base, _RPT)], idxbuf)
        for b in range(nmb):
            pltpu.async_copy(msgs.at[pl.ds((base + b * chk) * _IL, chk * _IL)],
                             mbs[b], lss[b])

        def outer(g, carry):
            for b in range(nmb):
                c = g * nmb + b
                pltpu.make_async_copy(
                    msgs.at[pl.ds((base + c * chk) * _IL, chk * _IL)],
                    mbs[b], lss[b]).wait()
                for r in range(chk):
                    pltpu.sync_copy(mbs[b].at[pl.ds(r * _IL, _IL)],
                                    acc.at[idxbuf.at[c * chk + r]], add=True)

                @pl.when(c < nchk - nmb)
                def _():
                    pltpu.async_copy(
                        msgs.at[pl.ds((base + (c + nmb) * chk) * _IL,
                                      chk * _IL)],
                        mbs[b], lss[b])

            return carry

        lax.fori_loop(0, ngrp2, outer, 0)
        plsc.subcore_barrier()
        pltpu.sync_copy(acc.at[pl.ds(sid * _RPS, _RPS)],
                        out.at[cid].at[pl.ds(sid * _RPS, _RPS)])

    return sc_gather, sc_scatter


def _pad_dw(v):
    return jnp.concatenate(
        [v, jnp.zeros((v.shape[0], _DW - _DIM), jnp.float32)], axis=1)


def _prep_body(x_ref, w_ref, b_ref, dp_ref, out_ref, inv_ref):
    o = _leaky(
        jnp.dot(x_ref[...], w_ref[...], preferred_element_type=jnp.float32)
        + b_ref[...])
    out_ref[...] = _pad_dw(o)
    deg = dp_ref[0, :, 0:_DIM] + dp_ref[1, :, 0:_DIM]
    inv_ref[...] = 1.0 / jnp.maximum(deg, 1.0)


def _h1_body(ea_ref, w_ref, b_ref, h1_ref):
    h1 = _leaky(
        jnp.dot(ea_ref[...], w_ref[...], preferred_element_type=jnp.float32)
        + b_ref[...])
    h1_ref[...] = h1.T


def _msg_body(h1t_ref, g_ref, w3t_ref, b2t_ref, msg_ref):
    h1t = h1t_ref[...]
    gt = g_ref[:, 0:_DIM].T
    pt = (jnp.reshape(jnp.broadcast_to(h1t[:, None, :], (_DIM, _DIM, _EB)),
                      (_DIM * _DIM, _EB))
          * jnp.reshape(jnp.broadcast_to(gt[None, :, :], (_DIM, _DIM, _EB)),
                        (_DIM * _DIM, _EB)))
    msgt = (jnp.dot(w3t_ref[...], pt, preferred_element_type=jnp.float32)
            + jnp.dot(b2t_ref[...], gt, preferred_element_type=jnp.float32))
    msg_ref[...] = _pad_dw(msgt.T)


def _node_body(dp, inv, out, h, rwt, cb, wirt, wizt, wint, whrt, whzt, whnt,
               bir, biz, binn, bhr, bhz, bhn, h_new):
    agg = (dp[0, :, 0:_DIM] + dp[1, :, 0:_DIM]) * inv[...]
    o = out[:, 0:_DIM]
    hh = h[:, 0:_DIM]
    m = _leaky(agg + jnp.dot(o, rwt[...], preferred_element_type=jnp.float32)
               + cb[...])
    gh_r = jnp.dot(hh, whrt[...], preferred_element_type=jnp.float32) + bhr[...]
    gh_z = jnp.dot(hh, whzt[...], preferred_element_type=jnp.float32) + bhz[...]
    gh_n = jnp.dot(hh, whnt[...], preferred_element_type=jnp.float32) + bhn[...]
    r = jax.nn.sigmoid(
        jnp.dot(m, wirt[...], preferred_element_type=jnp.float32) + bir[...]
        + gh_r)
    z = jax.nn.sigmoid(
        jnp.dot(m, wizt[...], preferred_element_type=jnp.float32) + biz[...]
        + gh_z)
    n = jnp.tanh(
        jnp.dot(m, wint[...], preferred_element_type=jnp.float32) + binn[...]
        + r * gh_n)
    h_new[...] = _pad_dw((1.0 - z) * n + z * hh)


def _s2s_body(out_ref, bcol_ref, brow_ref, gi_ref, gf_ref, gg_ref, go_ref,
              wq_ref, wr_ref, ob_ref, res_ref):
    o = out_ref[0:_N, 0:_DIM]
    bcol = bcol_ref[...]
    brow = brow_ref[...]
    i_g = jax.nn.sigmoid(gi_ref[...])
    f_g = jax.nn.sigmoid(gf_ref[...])
    g_g = jnp.tanh(gg_ref[...])
    o_g = jax.nn.sigmoid(go_ref[...])
    del f_g
    q = o_g * jnp.tanh(i_g * g_g)
    e = jnp.sum(o * q, axis=1, keepdims=True)
    iota_row = lax.broadcasted_iota(jnp.int32, (1, _B), 1)
    ohb = bcol == iota_row
    ohf = ohb.astype(jnp.float32)
    neg = jnp.float32(-jnp.inf)
    emax = jnp.max(jnp.where(ohb, e, neg), axis=0, keepdims=True)
    emax = jnp.where(jnp.isfinite(emax), emax, 0.0)
    emaxn = jnp.sum(ohf * emax, axis=1, keepdims=True)
    a = jnp.exp(e - emaxn)
    denom = jnp.sum(ohf * a, axis=0, keepdims=True)
    denomn = jnp.sum(ohf * jnp.maximum(denom, 1e-16), axis=1, keepdims=True)
    an = a / denomn
    iota_col = lax.broadcasted_iota(jnp.int32, (_B, 1), 0)
    oht = (brow == iota_col).astype(jnp.float32)
    rread = jnp.dot(oht, o * an, preferred_element_type=jnp.float32)
    res_ref[...] = (
        jnp.dot(jnp.broadcast_to(q, (_B, _DIM)), wq_ref[...],
                preferred_element_type=jnp.float32)
        + jnp.dot(rread, wr_ref[...], preferred_element_type=jnp.float32)
        + ob_ref[...])


def _full(shape):
    return pl.BlockSpec(shape, lambda *_: tuple(0 for _ in shape))


def kernel(x, edge_index, edge_attr, batch, lin0_W, lin0_b, net1_W, net1_b,
           net2_W, net2_b, root_W, conv_b, gru_Wih, gru_Whh, gru_bih, gru_bhh,
           lstm_Wih, lstm_Whh, lstm_bih, lstm_bhh, out_W, out_b):
    f32 = jnp.float32
    src = edge_index[0]
    dst = edge_index[1]
    srcp = jnp.pad(src, (0, _EP - _E)).reshape(_NIR, _IL)
    dstp = jnp.pad(dst, (0, _EP - _E), constant_values=_N).reshape(_NIR, _IL)
    eap = jnp.pad(edge_attr, ((0, _EP - _E), (0, 4)))
    xp = jnp.pad(x, ((0, _NP - _N), (0, 2)))

    lin0_Wt = jnp.pad(lin0_W.T, ((0, 2), (0, 0)))
    lin0_b2 = lin0_b.reshape(1, _DIM)
    net1_Wt = jnp.pad(net1_W.T, ((0, 4), (0, 0)))
    net1_b2 = net1_b.reshape(1, _DIM)
    w3t = net2_W.reshape(_DIM, _DIM, _DIM).transpose(2, 0, 1).reshape(
        _DIM * _DIM, _DIM).T
    b2t = net2_b.reshape(_DIM, _DIM).T
    root_Wt = root_W.T
    conv_b2 = conv_b.reshape(1, _DIM)
    wirt = gru_Wih[0:_DIM].T
    wizt = gru_Wih[_DIM:2 * _DIM].T
    wint = gru_Wih[2 * _DIM:].T
    whrt = gru_Whh[0:_DIM].T
    whzt = gru_Whh[_DIM:2 * _DIM].T
    whnt = gru_Whh[2 * _DIM:].T
    bir = gru_bih[0:_DIM].reshape(1, _DIM)
    biz = gru_bih[_DIM:2 * _DIM].reshape(1, _DIM)
    binn = gru_bih[2 * _DIM:].reshape(1, _DIM)
    bhr = gru_bhh[0:_DIM].reshape(1, _DIM)
    bhz = gru_bhh[_DIM:2 * _DIM].reshape(1, _DIM)
    bhn = gru_bhh[2 * _DIM:].reshape(1, _DIM)
    lstm_b = (lstm_bih + lstm_bhh)
    gi_b = lstm_b[0:_DIM].reshape(1, _DIM)
    gf_b = lstm_b[_DIM:2 * _DIM].reshape(1, _DIM)
    gg_b = lstm_b[2 * _DIM:3 * _DIM].reshape(1, _DIM)
    go_b = lstm_b[3 * _DIM:].reshape(1, _DIM)
    out_Wt = out_W.T
    wq = out_Wt[0:_DIM]
    wr = out_Wt[_DIM:]
    ob = out_b.reshape(1, 2)
    bcol = batch.reshape(_N, 1)
    brow = batch.reshape(1, _N)

    zeros_nd = jnp.zeros((_NP, _DW), f32)
    ones_e = jnp.ones((_EP, _DW), f32)

    sc_gather, sc_scatter = _sc_kernels()
    degp = sc_scatter(ones_e, dstp, zeros_nd)

    out0, invdeg = pl.pallas_call(
        _prep_body,
        in_specs=[_full((_NP, 16)), _full((16, _DIM)), _full((1, _DIM)),
                  _full((_NC, _NP, _DW))],
        out_specs=[_full((_NP, _DW)), _full((_NP, _DIM))],
        out_shape=[jax.ShapeDtypeStruct((_NP, _DW), f32),
                   jax.ShapeDtypeStruct((_NP, _DIM), f32)],
    )(xp, lin0_Wt, lin0_b2, degp)

    h1t = pl.pallas_call(
        _h1_body,
        grid=(_EP // _EB1,),
        in_specs=[pl.BlockSpec((_EB1, 8), lambda i: (i, 0)),
                  pl.BlockSpec((8, _DIM), lambda i: (0, 0)),
                  pl.BlockSpec((1, _DIM), lambda i: (0, 0))],
        out_specs=pl.BlockSpec((_DIM, _EB1), lambda i: (0, i)),
        out_shape=jax.ShapeDtypeStruct((_DIM, _EP), f32),
    )(eap, net1_Wt, net1_b2)

    msg_call = pl.pallas_call(
        _msg_body,
        grid=(_EP // _EB,),
        in_specs=[pl.BlockSpec((_DIM, _EB), lambda i: (0, i)),
                  pl.BlockSpec((_EB, _DW), lambda i: (i, 0)),
                  pl.BlockSpec((_DIM, _DIM * _DIM), lambda i: (0, 0)),
                  pl.BlockSpec((_DIM, _DIM), lambda i: (0, 0))],
        out_specs=pl.BlockSpec((_EB, _DW), lambda i: (i, 0)),
        out_shape=jax.ShapeDtypeStruct((_EP, _DW), f32),
    )

    node_call = pl.pallas_call(
        _node_body,
        in_specs=[_full((_NC, _NP, _DW)), _full((_NP, _DIM)),
                  _full((_NP, _DW)), _full((_NP, _DW))]
        + [_full((_DIM, _DIM))] + [_full((1, _DIM))]
        + [_full((_DIM, _DIM))] * 6 + [_full((1, _DIM))] * 6,
        out_specs=_full((_NP, _DW)),
        out_shape=jax.ShapeDtypeStruct((_NP, _DW), f32),
    )

    h = out0
    for _ in range(6):
        gath = sc_gather(h, srcp)
        msg = msg_call(h1t, gath, w3t, b2t)
        parts = sc_scatter(msg, dstp, zeros_nd)
        h = node_call(parts, invdeg, h, h, root_Wt, conv_b2,
                      wirt, wizt, wint, whrt, whzt, whnt,
                      bir, biz, binn, bhr, bhz, bhn)

    res = pl.pallas_call(
        _s2s_body,
        in_specs=[_full((_NP, _DW)), _full((_N, 1)), _full((1, _N))]
        + [_full((1, _DIM))] * 4 + [_full((_DIM, 2))] * 2 + [_full((1, 2))],
        out_specs=_full((_B, 2)),
        out_shape=jax.ShapeDtypeStruct((_B, 2), f32),
    )(h, bcol, brow, gi_b, gf_b, gg_b, go_b, wq, wr, ob)
    return res


# speedup vs baseline: 2.7931x; 1.0006x over previous
import functools

import jax
import jax.numpy as jnp
from jax import lax
from jax.experimental import pallas as pl
from jax.experimental.pallas import tpu as pltpu
from jax.experimental.pallas import tpu_sc as plsc

_N = 10000
_E = 160000
_DIM = 32
_B = 64

_NC = 2
_NS = 16
_NW = _NC * _NS
_IL = 128
_EP = 163840
_NIR = _EP // _IL
_RPT = _NIR // _NW
_NP = 10112
_RPS = _NP // _NS

_DW = 128
_EB = 512
_EB1 = 2048

def _leaky(v):
    return jnp.where(v >= 0, v, 0.01 * v)


@functools.cache
def _sc_kernels():
    mesh = plsc.VectorSubcoreMesh(core_axis_name="c", subcore_axis_name="s",
                                  num_cores=_NC, num_subcores=_NS)

    nbuf = 5
    ngrp = _RPT // nbuf

    @functools.partial(
        pl.kernel,
        out_type=jax.ShapeDtypeStruct((_EP, _DW), jnp.float32),
        mesh=mesh,
        scratch_types=[pltpu.VMEM((_RPT, _IL), jnp.int32)]
        + [pltpu.VMEM((_IL, _DW), jnp.float32)] * nbuf
        + [pltpu.SemaphoreType.DMA] * (2 * nbuf),
    )
    def sc_gather(table, idx, out, idxbuf, *bufs_sems):
        rbs = bufs_sems[:nbuf]
        gss = bufs_sems[nbuf:2 * nbuf]
        sss = bufs_sems[2 * nbuf:]
        wid = lax.axis_index("s") * _NC + lax.axis_index("c")
        base = wid * _RPT
        pltpu.sync_copy(idx.at[pl.ds(base, _RPT)], idxbuf)
        for b in range(nbuf):
            pltpu.async_copy(table.at[idxbuf.at[b]], rbs[b], gss[b])

        def outer(g, carry):
            for b in range(nbuf):
                j = g * nbuf + b
                pltpu.make_async_copy(table.at[idxbuf.at[b]], rbs[b],
                                      gss[b]).wait()
                pltpu.async_copy(rbs[b], out.at[pl.ds((base + j) * _IL, _IL)],
                                 sss[b])

            @pl.when(g < ngrp - 1)
            def _():
                for b in range(nbuf):
                    j = g * nbuf + b
                    pltpu.make_async_copy(
                        rbs[b], out.at[pl.ds((base + j) * _IL, _IL)],
                        sss[b]).wait()
                    pltpu.async_copy(table.at[idxbuf.at[(g + 1) * nbuf + b]],
                                     rbs[b], gss[b])

            return carry

        lax.fori_loop(0, ngrp, outer, 0)
        for b in range(nbuf):
            j = (ngrp - 1) * nbuf + b
            pltpu.make_async_copy(rbs[b], out.at[pl.ds((base + j) * _IL, _IL)],
                                  sss[b]).wait()

    nmb = 2
    ngrp2 = _RPT // nmb
    tail = _RPT - ngrp2 * nmb

    @functools.partial(
        pl.kernel,
        out_type=jax.ShapeDtypeStruct((_NC, _NP, _DW), jnp.float32),
        mesh=mesh,
        scratch_types=[pltpu.VMEM((_RPT, _IL), jnp.int32)]
        + [pltpu.VMEM((_IL, _DW), jnp.float32)] * nmb
        + [pltpu.SemaphoreType.DMA] * nmb
        + [pltpu.VMEM_SHARED((_NP, _DW), jnp.float32)],
    )
    def sc_scatter(msgs, idx, zeros, out, idxbuf, *bufs):
        mbs = bufs[:nmb]
        lss = bufs[nmb:2 * nmb]
        acc = bufs[2 * nmb]
        cid = lax.axis_index("c")
        sid = lax.axis_index("s")
        wid = sid * _NC + cid
        pltpu.sync_copy(zeros.at[pl.ds(sid * _RPS, _RPS)],
                        acc.at[pl.ds(sid * _RPS, _RPS)])
        plsc.subcore_barrier()
        base = wid * _RPT
        pltpu.sync_copy(idx.at[pl.ds(base, _RPT)], idxbuf)
        for b in range(nmb):
            pltpu.async_copy(msgs.at[pl.ds((base + b) * _IL, _IL)],
                             mbs[b], lss[b])

        def step(c, b):
            pltpu.make_async_copy(msgs.at[pl.ds((base + c) * _IL, _IL)],
                                  mbs[b], lss[b]).wait()
            pltpu.sync_copy(mbs[b], acc.at[idxbuf.at[c]], add=True)

            @pl.when(c < _RPT - nmb)
            def _():
                pltpu.async_copy(msgs.at[pl.ds((base + c + nmb) * _IL, _IL)],
                                 mbs[b], lss[b])

        def outer(g, carry):
            for b in range(nmb):
                step(g * nmb + b, b)
            return carry

        lax.fori_loop(0, ngrp2, outer, 0)
        for b in range(tail):
            step(ngrp2 * nmb + b, b)
        plsc.subcore_barrier()
        pltpu.sync_copy(acc.at[pl.ds(sid * _RPS, _RPS)],
                        out.at[cid].at[pl.ds(sid * _RPS, _RPS)])

    return sc_gather, sc_scatter


def _pad_dw(v):
    return jnp.concatenate(
        [v, jnp.zeros((v.shape[0], _DW - _DIM), jnp.float32)], axis=1)


def _prep_body(x_ref, w_ref, b_ref, dp_ref, out_ref, inv_ref):
    o = _leaky(
        jnp.dot(x_ref[...], w_ref[...], preferred_element_type=jnp.float32)
        + b_ref[...])
    out_ref[...] = _pad_dw(o)
    deg = dp_ref[0, :, 0:_DIM] + dp_ref[1, :, 0:_DIM]
    inv_ref[...] = 1.0 / jnp.maximum(deg, 1.0)


def _h1_body(ea_ref, w_ref, b_ref, h1_ref):
    h1 = _leaky(
        jnp.dot(ea_ref[...], w_ref[...], preferred_element_type=jnp.float32)
        + b_ref[...])
    h1_ref[...] = h1.T


def _msg_body(h1t_ref, g_ref, w3t_ref, b2t_ref, msg_ref):
    h1t = h1t_ref[...]
    gt = g_ref[:, 0:_DIM].T
    pt = (jnp.reshape(jnp.broadcast_to(h1t[:, None, :], (_DIM, _DIM, _EB)),
                      (_DIM * _DIM, _EB))
          * jnp.reshape(jnp.broadcast_to(gt[None, :, :], (_DIM, _DIM, _EB)),
                        (_DIM * _DIM, _EB)))
    msgt = (jnp.dot(w3t_ref[...], pt, preferred_element_type=jnp.float32)
            + jnp.dot(b2t_ref[...], gt, preferred_element_type=jnp.float32))
    msg_ref[...] = _pad_dw(msgt.T)


def _node_body(dp, inv, out, h, rwt, cb, wirt, wizt, wint, whrt, whzt, whnt,
               bir, biz, binn, bhr, bhz, bhn, h_new):
    agg = (dp[0, :, 0:_DIM] + dp[1, :, 0:_DIM]) * inv[...]
    o = out[:, 0:_DIM]
    hh = h[:, 0:_DIM]
    m = _leaky(agg + jnp.dot(o, rwt[...], preferred_element_type=jnp.float32)
               + cb[...])
    gh_r = jnp.dot(hh, whrt[...], preferred_element_type=jnp.float32) + bhr[...]
    gh_z = jnp.dot(hh, whzt[...], preferred_element_type=jnp.float32) + bhz[...]
    gh_n = jnp.dot(hh, whnt[...], preferred_element_type=jnp.float32) + bhn[...]
    r = jax.nn.sigmoid(
        jnp.dot(m, wirt[...], preferred_element_type=jnp.float32) + bir[...]
        + gh_r)
    z = jax.nn.sigmoid(
        jnp.dot(m, wizt[...], preferred_element_type=jnp.float32) + biz[...]
        + gh_z)
    n = jnp.tanh(
        jnp.dot(m, wint[...], preferred_element_type=jnp.float32) + binn[...]
        + r * gh_n)
    h_new[...] = _pad_dw((1.0 - z) * n + z * hh)


def _s2s_body(out_ref, bcol_ref, brow_ref, gi_ref, gf_ref, gg_ref, go_ref,
              wq_ref, wr_ref, ob_ref, res_ref):
    o = out_ref[0:_N, 0:_DIM]
    bcol = bcol_ref[...]
    brow = brow_ref[...]
    i_g = jax.nn.sigmoid(gi_ref[...])
    f_g = jax.nn.sigmoid(gf_ref[...])
    g_g = jnp.tanh(gg_ref[...])
    o_g = jax.nn.sigmoid(go_ref[...])
    del f_g
    q = o_g * jnp.tanh(i_g * g_g)
    e = jnp.sum(o * q, axis=1, keepdims=True)
    iota_row = lax.broadcasted_iota(jnp.int32, (1, _B), 1)
    ohb = bcol == iota_row
    ohf = ohb.astype(jnp.float32)
    neg = jnp.float32(-jnp.inf)
    emax = jnp.max(jnp.where(ohb, e, neg), axis=0, keepdims=True)
    emax = jnp.where(jnp.isfinite(emax), emax, 0.0)
    emaxn = jnp.sum(ohf * emax, axis=1, keepdims=True)
    a = jnp.exp(e - emaxn)
    denom = jnp.sum(ohf * a, axis=0, keepdims=True)
    denomn = jnp.sum(ohf * jnp.maximum(denom, 1e-16), axis=1, keepdims=True)
    an = a / denomn
    iota_col = lax.broadcasted_iota(jnp.int32, (_B, 1), 0)
    oht = (brow == iota_col).astype(jnp.float32)
    rread = jnp.dot(oht, o * an, preferred_element_type=jnp.float32)
    res_ref[...] = (
        jnp.dot(jnp.broadcast_to(q, (_B, _DIM)), wq_ref[...],
                preferred_element_type=jnp.float32)
        + jnp.dot(rread, wr_ref[...], preferred_element_type=jnp.float32)
        + ob_ref[...])


def _full(shape):
    return pl.BlockSpec(shape, lambda *_: tuple(0 for _ in shape))


def kernel(x, edge_index, edge_attr, batch, lin0_W, lin0_b, net1_W, net1_b,
           net2_W, net2_b, root_W, conv_b, gru_Wih, gru_Whh, gru_bih, gru_bhh,
           lstm_Wih, lstm_Whh, lstm_bih, lstm_bhh, out_W, out_b):
    f32 = jnp.float32
    src = edge_index[0]
    dst = edge_index[1]
    srcp = jnp.pad(src, (0, _EP - _E)).reshape(_NIR, _IL)
    dstp = jnp.pad(dst, (0, _EP - _E), constant_values=_N).reshape(_NIR, _IL)
    eap = jnp.pad(edge_attr, ((0, _EP - _E), (0, 4)))
    xp = jnp.pad(x, ((0, _NP - _N), (0, 2)))

    lin0_Wt = jnp.pad(lin0_W.T, ((0, 2), (0, 0)))
    lin0_b2 = lin0_b.reshape(1, _DIM)
    net1_Wt = jnp.pad(net1_W.T, ((0, 4), (0, 0)))
    net1_b2 = net1_b.reshape(1, _DIM)
    w3t = net2_W.reshape(_DIM, _DIM, _DIM).transpose(2, 0, 1).reshape(
        _DIM * _DIM, _DIM).T
    b2t = net2_b.reshape(_DIM, _DIM).T
    root_Wt = root_W.T
    conv_b2 = conv_b.reshape(1, _DIM)
    wirt = gru_Wih[0:_DIM].T
    wizt = gru_Wih[_DIM:2 * _DIM].T
    wint = gru_Wih[2 * _DIM:].T
    whrt = gru_Whh[0:_DIM].T
    whzt = gru_Whh[_DIM:2 * _DIM].T
    whnt = gru_Whh[2 * _DIM:].T
    bir = gru_bih[0:_DIM].reshape(1, _DIM)
    biz = gru_bih[_DIM:2 * _DIM].reshape(1, _DIM)
    binn = gru_bih[2 * _DIM:].reshape(1, _DIM)
    bhr = gru_bhh[0:_DIM].reshape(1, _DIM)
    bhz = gru_bhh[_DIM:2 * _DIM].reshape(1, _DIM)
    bhn = gru_bhh[2 * _DIM:].reshape(1, _DIM)
    lstm_b = (lstm_bih + lstm_bhh)
    gi_b = lstm_b[0:_DIM].reshape(1, _DIM)
    gf_b = lstm_b[_DIM:2 * _DIM].reshape(1, _DIM)
    gg_b = lstm_b[2 * _DIM:3 * _DIM].reshape(1, _DIM)
    go_b = lstm_b[3 * _DIM:].reshape(1, _DIM)
    out_Wt = out_W.T
    wq = out_Wt[0:_DIM]
    wr = out_Wt[_DIM:]
    ob = out_b.reshape(1, 2)
    bcol = batch.reshape(_N, 1)
    brow = batch.reshape(1, _N)

    zeros_nd = jnp.zeros((_NP, _DW), f32)
    ones_e = jnp.ones((_EP, _DW), f32)

    sc_gather, sc_scatter = _sc_kernels()
    degp = sc_scatter(ones_e, dstp, zeros_nd)

    out0, invdeg = pl.pallas_call(
        _prep_body,
        in_specs=[_full((_NP, 16)), _full((16, _DIM)), _full((1, _DIM)),
                  _full((_NC, _NP, _DW))],
        out_specs=[_full((_NP, _DW)), _full((_NP, _DIM))],
        out_shape=[jax.ShapeDtypeStruct((_NP, _DW), f32),
                   jax.ShapeDtypeStruct((_NP, _DIM), f32)],
    )(xp, lin0_Wt, lin0_b2, degp)

    h1t = pl.pallas_call(
        _h1_body,
        grid=(_EP // _EB1,),
        in_specs=[pl.BlockSpec((_EB1, 8), lambda i: (i, 0)),
                  pl.BlockSpec((8, _DIM), lambda i: (0, 0)),
                  pl.BlockSpec((1, _DIM), lambda i: (0, 0))],
        out_specs=pl.BlockSpec((_DIM, _EB1), lambda i: (0, i)),
        out_shape=jax.ShapeDtypeStruct((_DIM, _EP), f32),
    )(eap, net1_Wt, net1_b2)

    msg_call = pl.pallas_call(
        _msg_body,
        grid=(_EP // _EB,),
        in_specs=[pl.BlockSpec((_DIM, _EB), lambda i: (0, i)),
                  pl.BlockSpec((_EB, _DW), lambda i: (i, 0)),
                  pl.BlockSpec((_DIM, _DIM * _DIM), lambda i: (0, 0)),
                  pl.BlockSpec((_DIM, _DIM), lambda i: (0, 0))],
        out_specs=pl.BlockSpec((_EB, _DW), lambda i: (i, 0)),
        out_shape=jax.ShapeDtypeStruct((_EP, _DW), f32),
    )

    node_call = pl.pallas_call(
        _node_body,
        in_specs=[_full((_NC, _NP, _DW)), _full((_NP, _DIM)),
                  _full((_NP, _DW)), _full((_NP, _DW))]
        + [_full((_DIM, _DIM))] + [_full((1, _DIM))]
        + [_full((_DIM, _DIM))] * 6 + [_full((1, _DIM))] * 6,
        out_specs=_full((_NP, _DW)),
        out_shape=jax.ShapeDtypeStruct((_NP, _DW), f32),
    )

    h = out0
    for _ in range(6):
        gath = sc_gather(h, srcp)
        msg = msg_call(h1t, gath, w3t, b2t)
        parts = sc_scatter(msg, dstp, zeros_nd)
        h = node_call(parts, invdeg, h, h, root_Wt, conv_b2,
                      wirt, wizt, wint, whrt, whzt, whnt,
                      bir, biz, binn, bhr, bhz, bhn)

    res = pl.pallas_call(
        _s2s_body,
        in_specs=[_full((_NP, _DW)), _full((_N, 1)), _full((1, _N))]
        + [_full((1, _DIM))] * 4 + [_full((_DIM, 2))] * 2 + [_full((1, 2))],
        out_specs=_full((_B, 2)),
        out_shape=jax.ShapeDtypeStruct((_B, 2), f32),
    )(h, bcol, brow, gi_b, gf_b, gg_b, go_b, wq, wr, ob)
    return res


# speedup vs baseline: 4.0991x; 1.4675x over previous
import functools

import jax
import jax.numpy as jnp
from jax import lax
from jax.experimental import pallas as pl
from jax.experimental.pallas import tpu as pltpu
from jax.experimental.pallas import tpu_sc as plsc

_N = 10000
_E = 160000
_DIM = 32
_B = 64

_NC = 2
_NS = 16
_NW = _NC * _NS
_IL = 128
_EP = 163840
_NIR = _EP // _IL
_RPT = _NIR // _NW
_NP = 10112
_RPS = _NP // _NS

_DW = 128
_EB = 512
_EB1 = 2048

def _leaky(v):
    return jnp.where(v >= 0, v, 0.01 * v)


@functools.cache
def _sc_kernels():
    mesh = plsc.VectorSubcoreMesh(core_axis_name="c", subcore_axis_name="s",
                                  num_cores=_NC, num_subcores=_NS)

    nbuf = 2
    ngrp = _RPT // nbuf

    @functools.partial(
        pl.kernel,
        out_type=jax.ShapeDtypeStruct((_EP, _DW), jnp.float32),
        mesh=mesh,
        scratch_types=[pltpu.VMEM((_RPT, _IL), jnp.int32)]
        + [pltpu.VMEM((_IL, _DW), jnp.float32)] * nbuf
        + [pltpu.SemaphoreType.DMA] * (2 * nbuf)
        + [pltpu.VMEM_SHARED((_NP, _DW), jnp.float32)],
    )
    def sc_gather(table, idx, out, idxbuf, *bufs_sems):
        rbs = bufs_sems[:nbuf]
        gss = bufs_sems[nbuf:2 * nbuf]
        sss = bufs_sems[2 * nbuf:2 * nbuf + nbuf]
        tab = bufs_sems[2 * nbuf + nbuf]
        sid = lax.axis_index("s")
        wid = sid * _NC + lax.axis_index("c")
        base = wid * _RPT
        pltpu.sync_copy(table.at[pl.ds(sid * _RPS, _RPS)],
                        tab.at[pl.ds(sid * _RPS, _RPS)])
        pltpu.sync_copy(idx.at[pl.ds(base, _RPT)], idxbuf)
        plsc.subcore_barrier()
        for b in range(nbuf):
            pltpu.async_copy(tab.at[idxbuf.at[b]], rbs[b], gss[b])

        def outer(g, carry):
            for b in range(nbuf):
                j = g * nbuf + b
                pltpu.make_async_copy(tab.at[idxbuf.at[b]], rbs[b],
                                      gss[b]).wait()
                pltpu.async_copy(rbs[b], out.at[pl.ds((base + j) * _IL, _IL)],
                                 sss[b])

            @pl.when(g < ngrp - 1)
            def _():
                for b in range(nbuf):
                    j = g * nbuf + b
                    pltpu.make_async_copy(
                        rbs[b], out.at[pl.ds((base + j) * _IL, _IL)],
                        sss[b]).wait()
                    pltpu.async_copy(tab.at[idxbuf.at[(g + 1) * nbuf + b]],
                                     rbs[b], gss[b])

            return carry

        lax.fori_loop(0, ngrp, outer, 0)
        for b in range(nbuf):
            j = (ngrp - 1) * nbuf + b
            pltpu.make_async_copy(rbs[b], out.at[pl.ds((base + j) * _IL, _IL)],
                                  sss[b]).wait()

    nmb = 2
    ngrp2 = _RPT // nmb
    tail = _RPT - ngrp2 * nmb

    @functools.partial(
        pl.kernel,
        out_type=jax.ShapeDtypeStruct((_NC, _NP, _DW), jnp.float32),
        mesh=mesh,
        scratch_types=[pltpu.VMEM((_RPT, _IL), jnp.int32)]
        + [pltpu.VMEM((_IL, _DW), jnp.float32)] * nmb
        + [pltpu.SemaphoreType.DMA] * nmb
        + [pltpu.VMEM_SHARED((_NP, _DW), jnp.float32)],
    )
    def sc_scatter(msgs, idx, zeros, out, idxbuf, *bufs):
        mbs = bufs[:nmb]
        lss = bufs[nmb:2 * nmb]
        acc = bufs[2 * nmb]
        cid = lax.axis_index("c")
        sid = lax.axis_index("s")
        wid = sid * _NC + cid
        pltpu.sync_copy(zeros.at[pl.ds(sid * _RPS, _RPS)],
                        acc.at[pl.ds(sid * _RPS, _RPS)])
        plsc.subcore_barrier()
        base = wid * _RPT
        pltpu.sync_copy(idx.at[pl.ds(base, _RPT)], idxbuf)
        for b in range(nmb):
            pltpu.async_copy(msgs.at[pl.ds((base + b) * _IL, _IL)],
                             mbs[b], lss[b])

        def step(c, b):
            pltpu.make_async_copy(msgs.at[pl.ds((base + c) * _IL, _IL)],
                                  mbs[b], lss[b]).wait()
            pltpu.sync_copy(mbs[b], acc.at[idxbuf.at[c]], add=True)

            @pl.when(c < _RPT - nmb)
            def _():
                pltpu.async_copy(msgs.at[pl.ds((base + c + nmb) * _IL, _IL)],
                                 mbs[b], lss[b])

        def outer(g, carry):
            for b in range(nmb):
                step(g * nmb + b, b)
            return carry

        lax.fori_loop(0, ngrp2, outer, 0)
        for b in range(tail):
            step(ngrp2 * nmb + b, b)
        plsc.subcore_barrier()
        pltpu.sync_copy(acc.at[pl.ds(sid * _RPS, _RPS)],
                        out.at[cid].at[pl.ds(sid * _RPS, _RPS)])

    return sc_gather, sc_scatter


def _pad_dw(v):
    return jnp.concatenate(
        [v, jnp.zeros((v.shape[0], _DW - _DIM), jnp.float32)], axis=1)


def _prep_body(x_ref, w_ref, b_ref, dp_ref, out_ref, inv_ref):
    o = _leaky(
        jnp.dot(x_ref[...], w_ref[...], preferred_element_type=jnp.float32)
        + b_ref[...])
    out_ref[...] = _pad_dw(o)
    deg = dp_ref[0, :, 0:_DIM] + dp_ref[1, :, 0:_DIM]
    inv_ref[...] = 1.0 / jnp.maximum(deg, 1.0)


def _h1_body(ea_ref, w_ref, b_ref, h1_ref):
    h1 = _leaky(
        jnp.dot(ea_ref[...], w_ref[...], preferred_element_type=jnp.float32)
        + b_ref[...])
    h1_ref[...] = h1.T


def _msg_body(h1t_ref, g_ref, w3t_ref, b2t_ref, msg_ref):
    h1t = h1t_ref[...]
    gt = g_ref[:, 0:_DIM].T
    pt = (jnp.reshape(jnp.broadcast_to(h1t[:, None, :], (_DIM, _DIM, _EB)),
                      (_DIM * _DIM, _EB))
          * jnp.reshape(jnp.broadcast_to(gt[None, :, :], (_DIM, _DIM, _EB)),
                        (_DIM * _DIM, _EB)))
    msgt = (jnp.dot(w3t_ref[...], pt, preferred_element_type=jnp.float32)
            + jnp.dot(b2t_ref[...], gt, preferred_element_type=jnp.float32))
    msg_ref[...] = _pad_dw(msgt.T)


def _node_body(dp, inv, out, h, rwt, cb, wirt, wizt, wint, whrt, whzt, whnt,
               bir, biz, binn, bhr, bhz, bhn, h_new):
    agg = (dp[0, :, 0:_DIM] + dp[1, :, 0:_DIM]) * inv[...]
    o = out[:, 0:_DIM]
    hh = h[:, 0:_DIM]
    m = _leaky(agg + jnp.dot(o, rwt[...], preferred_element_type=jnp.float32)
               + cb[...])
    gh_r = jnp.dot(hh, whrt[...], preferred_element_type=jnp.float32) + bhr[...]
    gh_z = jnp.dot(hh, whzt[...], preferred_element_type=jnp.float32) + bhz[...]
    gh_n = jnp.dot(hh, whnt[...], preferred_element_type=jnp.float32) + bhn[...]
    r = jax.nn.sigmoid(
        jnp.dot(m, wirt[...], preferred_element_type=jnp.float32) + bir[...]
        + gh_r)
    z = jax.nn.sigmoid(
        jnp.dot(m, wizt[...], preferred_element_type=jnp.float32) + biz[...]
        + gh_z)
    n = jnp.tanh(
        jnp.dot(m, wint[...], preferred_element_type=jnp.float32) + binn[...]
        + r * gh_n)
    h_new[...] = _pad_dw((1.0 - z) * n + z * hh)


def _s2s_body(out_ref, bcol_ref, brow_ref, gi_ref, gf_ref, gg_ref, go_ref,
              wq_ref, wr_ref, ob_ref, res_ref):
    o = out_ref[0:_N, 0:_DIM]
    bcol = bcol_ref[...]
    brow = brow_ref[...]
    i_g = jax.nn.sigmoid(gi_ref[...])
    f_g = jax.nn.sigmoid(gf_ref[...])
    g_g = jnp.tanh(gg_ref[...])
    o_g = jax.nn.sigmoid(go_ref[...])
    del f_g
    q = o_g * jnp.tanh(i_g * g_g)
    e = jnp.sum(o * q, axis=1, keepdims=True)
    iota_row = lax.broadcasted_iota(jnp.int32, (1, _B), 1)
    ohb = bcol == iota_row
    ohf = ohb.astype(jnp.float32)
    neg = jnp.float32(-jnp.inf)
    emax = jnp.max(jnp.where(ohb, e, neg), axis=0, keepdims=True)
    emax = jnp.where(jnp.isfinite(emax), emax, 0.0)
    emaxn = jnp.sum(ohf * emax, axis=1, keepdims=True)
    a = jnp.exp(e - emaxn)
    denom = jnp.sum(ohf * a, axis=0, keepdims=True)
    denomn = jnp.sum(ohf * jnp.maximum(denom, 1e-16), axis=1, keepdims=True)
    an = a / denomn
    iota_col = lax.broadcasted_iota(jnp.int32, (_B, 1), 0)
    oht = (brow == iota_col).astype(jnp.float32)
    rread = jnp.dot(oht, o * an, preferred_element_type=jnp.float32)
    res_ref[...] = (
        jnp.dot(jnp.broadcast_to(q, (_B, _DIM)), wq_ref[...],
                preferred_element_type=jnp.float32)
        + jnp.dot(rread, wr_ref[...], preferred_element_type=jnp.float32)
        + ob_ref[...])


def _full(shape):
    return pl.BlockSpec(shape, lambda *_: tuple(0 for _ in shape))


def kernel(x, edge_index, edge_attr, batch, lin0_W, lin0_b, net1_W, net1_b,
           net2_W, net2_b, root_W, conv_b, gru_Wih, gru_Whh, gru_bih, gru_bhh,
           lstm_Wih, lstm_Whh, lstm_bih, lstm_bhh, out_W, out_b):
    f32 = jnp.float32
    src = edge_index[0]
    dst = edge_index[1]
    srcp = jnp.pad(src, (0, _EP - _E)).reshape(_NIR, _IL)
    dstp = jnp.pad(dst, (0, _EP - _E), constant_values=_N).reshape(_NIR, _IL)
    eap = jnp.pad(edge_attr, ((0, _EP - _E), (0, 4)))
    xp = jnp.pad(x, ((0, _NP - _N), (0, 2)))

    lin0_Wt = jnp.pad(lin0_W.T, ((0, 2), (0, 0)))
    lin0_b2 = lin0_b.reshape(1, _DIM)
    net1_Wt = jnp.pad(net1_W.T, ((0, 4), (0, 0)))
    net1_b2 = net1_b.reshape(1, _DIM)
    w3t = net2_W.reshape(_DIM, _DIM, _DIM).transpose(2, 0, 1).reshape(
        _DIM * _DIM, _DIM).T
    b2t = net2_b.reshape(_DIM, _DIM).T
    root_Wt = root_W.T
    conv_b2 = conv_b.reshape(1, _DIM)
    wirt = gru_Wih[0:_DIM].T
    wizt = gru_Wih[_DIM:2 * _DIM].T
    wint = gru_Wih[2 * _DIM:].T
    whrt = gru_Whh[0:_DIM].T
    whzt = gru_Whh[_DIM:2 * _DIM].T
    whnt = gru_Whh[2 * _DIM:].T
    bir = gru_bih[0:_DIM].reshape(1, _DIM)
    biz = gru_bih[_DIM:2 * _DIM].reshape(1, _DIM)
    binn = gru_bih[2 * _DIM:].reshape(1, _DIM)
    bhr = gru_bhh[0:_DIM].reshape(1, _DIM)
    bhz = gru_bhh[_DIM:2 * _DIM].reshape(1, _DIM)
    bhn = gru_bhh[2 * _DIM:].reshape(1, _DIM)
    lstm_b = (lstm_bih + lstm_bhh)
    gi_b = lstm_b[0:_DIM].reshape(1, _DIM)
    gf_b = lstm_b[_DIM:2 * _DIM].reshape(1, _DIM)
    gg_b = lstm_b[2 * _DIM:3 * _DIM].reshape(1, _DIM)
    go_b = lstm_b[3 * _DIM:].reshape(1, _DIM)
    out_Wt = out_W.T
    wq = out_Wt[0:_DIM]
    wr = out_Wt[_DIM:]
    ob = out_b.reshape(1, 2)
    bcol = batch.reshape(_N, 1)
    brow = batch.reshape(1, _N)

    zeros_nd = jnp.zeros((_NP, _DW), f32)
    ones_e = jnp.ones((_EP, _DW), f32)

    sc_gather, sc_scatter = _sc_kernels()
    degp = sc_scatter(ones_e, dstp, zeros_nd)

    out0, invdeg = pl.pallas_call(
        _prep_body,
        in_specs=[_full((_NP, 16)), _full((16, _DIM)), _full((1, _DIM)),
                  _full((_NC, _NP, _DW))],
        out_specs=[_full((_NP, _DW)), _full((_NP, _DIM))],
        out_shape=[jax.ShapeDtypeStruct((_NP, _DW), f32),
                   jax.ShapeDtypeStruct((_NP, _DIM), f32)],
    )(xp, lin0_Wt, lin0_b2, degp)

    h1t = pl.pallas_call(
        _h1_body,
        grid=(_EP // _EB1,),
        in_specs=[pl.BlockSpec((_EB1, 8), lambda i: (i, 0)),
                  pl.BlockSpec((8, _DIM), lambda i: (0, 0)),
                  pl.BlockSpec((1, _DIM), lambda i: (0, 0))],
        out_specs=pl.BlockSpec((_DIM, _EB1), lambda i: (0, i)),
        out_shape=jax.ShapeDtypeStruct((_DIM, _EP), f32),
    )(eap, net1_Wt, net1_b2)

    msg_call = pl.pallas_call(
        _msg_body,
        grid=(_EP // _EB,),
        in_specs=[pl.BlockSpec((_DIM, _EB), lambda i: (0, i)),
                  pl.BlockSpec((_EB, _DW), lambda i: (i, 0)),
                  pl.BlockSpec((_DIM, _DIM * _DIM), lambda i: (0, 0)),
                  pl.BlockSpec((_DIM, _DIM), lambda i: (0, 0))],
        out_specs=pl.BlockSpec((_EB, _DW), lambda i: (i, 0)),
        out_shape=jax.ShapeDtypeStruct((_EP, _DW), f32),
    )

    node_call = pl.pallas_call(
        _node_body,
        in_specs=[_full((_NC, _NP, _DW)), _full((_NP, _DIM)),
                  _full((_NP, _DW)), _full((_NP, _DW))]
        + [_full((_DIM, _DIM))] + [_full((1, _DIM))]
        + [_full((_DIM, _DIM))] * 6 + [_full((1, _DIM))] * 6,
        out_specs=_full((_NP, _DW)),
        out_shape=jax.ShapeDtypeStruct((_NP, _DW), f32),
    )

    h = out0
    for _ in range(6):
        gath = sc_gather(h, srcp)
        msg = msg_call(h1t, gath, w3t, b2t)
        parts = sc_scatter(msg, dstp, zeros_nd)
        h = node_call(parts, invdeg, h, h, root_Wt, conv_b2,
                      wirt, wizt, wint, whrt, whzt, whnt,
                      bir, biz, binn, bhr, bhz, bhn)

    res = pl.pallas_call(
        _s2s_body,
        in_specs=[_full((_NP, _DW)), _full((_N, 1)), _full((1, _N))]
        + [_full((1, _DIM))] * 4 + [_full((_DIM, 2))] * 2 + [_full((1, 2))],
        out_specs=_full((_B, 2)),
        out_shape=jax.ShapeDtypeStruct((_B, 2), f32),
    )(h, bcol, brow, gi_b, gf_b, gg_b, go_b, wq, wr, ob)
    return res


# speedup vs baseline: 4.3788x; 1.0683x over previous
import functools

import jax
import jax.numpy as jnp
from jax import lax
from jax.experimental import pallas as pl
from jax.experimental.pallas import tpu as pltpu
from jax.experimental.pallas import tpu_sc as plsc

_N = 10000
_E = 160000
_DIM = 32
_B = 64

_NC = 2
_NS = 16
_NW = _NC * _NS
_IL = 128
_EP = 163840
_NIR = _EP // _IL
_RPT = _NIR // _NW
_NP = 10112
_RPS = _NP // _NS

_DW = 128
_EPH = _EP // 2
_EB = 512
_EB1 = 2048

def _leaky(v):
    return jnp.where(v >= 0, v, 0.01 * v)


@functools.cache
def _sc_kernels(nir):
    mesh = plsc.VectorSubcoreMesh(core_axis_name="c", subcore_axis_name="s",
                                  num_cores=_NC, num_subcores=_NS)
    rpt = nir // _NW
    nep = nir * _IL
    nld = rpt if rpt % 8 == 0 else rpt + 8 - rpt % 8

    def _idx_window(idx, idxbuf, wid):
        base = wid * rpt
        off = base % 8 if rpt % 8 else 0
        start = pl.multiple_of(base - off, 8)
        pltpu.sync_copy(idx.at[pl.ds(start, nld)], idxbuf)
        return off

    nbuf = 2
    ngrp = rpt // nbuf

    @functools.partial(
        pl.kernel,
        out_type=jax.ShapeDtypeStruct((nep, _DW), jnp.float32),
        mesh=mesh,
        scratch_types=[pltpu.VMEM((nld, _IL), jnp.int32)]
        + [pltpu.VMEM((_IL, _DW), jnp.float32)] * nbuf
        + [pltpu.SemaphoreType.DMA] * (2 * nbuf)
        + [pltpu.VMEM_SHARED((_NP, _DW), jnp.float32)],
    )
    def sc_gather(table, idx, out, idxbuf, *bufs_sems):
        rbs = bufs_sems[:nbuf]
        gss = bufs_sems[nbuf:2 * nbuf]
        sss = bufs_sems[2 * nbuf:2 * nbuf + nbuf]
        tab = bufs_sems[2 * nbuf + nbuf]
        sid = lax.axis_index("s")
        wid = sid * _NC + lax.axis_index("c")
        base = wid * rpt
        pltpu.sync_copy(table.at[pl.ds(sid * _RPS, _RPS)],
                        tab.at[pl.ds(sid * _RPS, _RPS)])
        off = _idx_window(idx, idxbuf, wid)
        plsc.subcore_barrier()
        for b in range(nbuf):
            pltpu.async_copy(tab.at[idxbuf.at[off + b]], rbs[b], gss[b])

        def outer(g, carry):
            for b in range(nbuf):
                j = g * nbuf + b
                pltpu.make_async_copy(tab.at[idxbuf.at[off + b]], rbs[b],
                                      gss[b]).wait()
                pltpu.async_copy(rbs[b], out.at[pl.ds((base + j) * _IL, _IL)],
                                 sss[b])

            @pl.when(g < ngrp - 1)
            def _():
                for b in range(nbuf):
                    j = g * nbuf + b
                    pltpu.make_async_copy(
                        rbs[b], out.at[pl.ds((base + j) * _IL, _IL)],
                        sss[b]).wait()
                    pltpu.async_copy(tab.at[idxbuf.at[off + (g + 1) * nbuf + b]],
                                     rbs[b], gss[b])

            return carry

        lax.fori_loop(0, ngrp, outer, 0)
        for b in range(nbuf):
            j = (ngrp - 1) * nbuf + b
            pltpu.make_async_copy(rbs[b], out.at[pl.ds((base + j) * _IL, _IL)],
                                  sss[b]).wait()

    nmb = 2
    ngrp2 = rpt // nmb
    tail = rpt - ngrp2 * nmb

    @functools.partial(
        pl.kernel,
        out_type=jax.ShapeDtypeStruct((_NC, _NP, _DW), jnp.float32),
        mesh=mesh,
        scratch_types=[pltpu.VMEM((nld, _IL), jnp.int32)]
        + [pltpu.VMEM((_IL, _DW), jnp.float32)] * nmb
        + [pltpu.SemaphoreType.DMA] * nmb
        + [pltpu.VMEM_SHARED((_NP, _DW), jnp.float32)],
    )
    def sc_scatter(msgs, idx, zeros, out, idxbuf, *bufs):
        mbs = bufs[:nmb]
        lss = bufs[nmb:2 * nmb]
        acc = bufs[2 * nmb]
        cid = lax.axis_index("c")
        sid = lax.axis_index("s")
        wid = sid * _NC + cid
        pltpu.sync_copy(zeros.at[pl.ds(sid * _RPS, _RPS)],
                        acc.at[pl.ds(sid * _RPS, _RPS)])
        plsc.subcore_barrier()
        base = wid * rpt
        off = _idx_window(idx, idxbuf, wid)
        for b in range(nmb):
            pltpu.async_copy(msgs.at[pl.ds((base + b) * _IL, _IL)],
                             mbs[b], lss[b])

        def step(c, b):
            pltpu.make_async_copy(msgs.at[pl.ds((base + c) * _IL, _IL)],
                                  mbs[b], lss[b]).wait()
            pltpu.sync_copy(mbs[b], acc.at[idxbuf.at[off + c]], add=True)

            @pl.when(c < rpt - nmb)
            def _():
                pltpu.async_copy(msgs.at[pl.ds((base + c + nmb) * _IL, _IL)],
                                 mbs[b], lss[b])

        def outer(g, carry):
            for b in range(nmb):
                step(g * nmb + b, b)
            return carry

        lax.fori_loop(0, ngrp2, outer, 0)
        for b in range(tail):
            step(ngrp2 * nmb + b, b)
        plsc.subcore_barrier()
        pltpu.sync_copy(acc.at[pl.ds(sid * _RPS, _RPS)],
                        out.at[cid].at[pl.ds(sid * _RPS, _RPS)])

    return sc_gather, sc_scatter


def _pad_dw(v):
    return jnp.concatenate(
        [v, jnp.zeros((v.shape[0], _DW - _DIM), jnp.float32)], axis=1)


def _prep_body(x_ref, w_ref, b_ref, dp_ref, out_ref, inv_ref):
    o = _leaky(
        jnp.dot(x_ref[...], w_ref[...], preferred_element_type=jnp.float32)
        + b_ref[...])
    out_ref[...] = _pad_dw(o)
    deg = dp_ref[0, :, 0:_DIM] + dp_ref[1, :, 0:_DIM]
    inv_ref[...] = 1.0 / jnp.maximum(deg, 1.0)


def _h1_body(ea_ref, w_ref, b_ref, h1_ref):
    h1 = _leaky(
        jnp.dot(ea_ref[...], w_ref[...], preferred_element_type=jnp.float32)
        + b_ref[...])
    h1_ref[...] = h1.T


def _msg_body(h1t_ref, g_ref, w3t_ref, b2t_ref, msg_ref):
    h1t = h1t_ref[...]
    gt = g_ref[:, 0:_DIM].T
    pt = (jnp.reshape(jnp.broadcast_to(h1t[:, None, :], (_DIM, _DIM, _EB)),
                      (_DIM * _DIM, _EB))
          * jnp.reshape(jnp.broadcast_to(gt[None, :, :], (_DIM, _DIM, _EB)),
                        (_DIM * _DIM, _EB)))
    msgt = (jnp.dot(w3t_ref[...], pt, preferred_element_type=jnp.float32)
            + jnp.dot(b2t_ref[...], gt, preferred_element_type=jnp.float32))
    msg_ref[...] = _pad_dw(msgt.T)


def _node_body(dpa, dpb, inv, out, h, rwt, cb, wirt, wizt, wint, whrt, whzt,
               whnt, bir, biz, binn, bhr, bhz, bhn, h_new):
    agg = (dpa[0, :, 0:_DIM] + dpa[1, :, 0:_DIM]
           + dpb[0, :, 0:_DIM] + dpb[1, :, 0:_DIM]) * inv[...]
    o = out[:, 0:_DIM]
    hh = h[:, 0:_DIM]
    m = _leaky(agg + jnp.dot(o, rwt[...], preferred_element_type=jnp.float32)
               + cb[...])
    gh_r = jnp.dot(hh, whrt[...], preferred_element_type=jnp.float32) + bhr[...]
    gh_z = jnp.dot(hh, whzt[...], preferred_element_type=jnp.float32) + bhz[...]
    gh_n = jnp.dot(hh, whnt[...], preferred_element_type=jnp.float32) + bhn[...]
    r = jax.nn.sigmoid(
        jnp.dot(m, wirt[...], preferred_element_type=jnp.float32) + bir[...]
        + gh_r)
    z = jax.nn.sigmoid(
        jnp.dot(m, wizt[...], preferred_element_type=jnp.float32) + biz[...]
        + gh_z)
    n = jnp.tanh(
        jnp.dot(m, wint[...], preferred_element_type=jnp.float32) + binn[...]
        + r * gh_n)
    h_new[...] = _pad_dw((1.0 - z) * n + z * hh)


def _s2s_body(out_ref, bcol_ref, brow_ref, gi_ref, gf_ref, gg_ref, go_ref,
              wq_ref, wr_ref, ob_ref, res_ref):
    o = out_ref[0:_N, 0:_DIM]
    bcol = bcol_ref[...]
    brow = brow_ref[...]
    i_g = jax.nn.sigmoid(gi_ref[...])
    f_g = jax.nn.sigmoid(gf_ref[...])
    g_g = jnp.tanh(gg_ref[...])
    o_g = jax.nn.sigmoid(go_ref[...])
    del f_g
    q = o_g * jnp.tanh(i_g * g_g)
    e = jnp.sum(o * q, axis=1, keepdims=True)
    iota_row = lax.broadcasted_iota(jnp.int32, (1, _B), 1)
    ohb = bcol == iota_row
    ohf = ohb.astype(jnp.float32)
    neg = jnp.float32(-jnp.inf)
    emax = jnp.max(jnp.where(ohb, e, neg), axis=0, keepdims=True)
    emax = jnp.where(jnp.isfinite(emax), emax, 0.0)
    emaxn = jnp.sum(ohf * emax, axis=1, keepdims=True)
    a = jnp.exp(e - emaxn)
    denom = jnp.sum(ohf * a, axis=0, keepdims=True)
    denomn = jnp.sum(ohf * jnp.maximum(denom, 1e-16), axis=1, keepdims=True)
    an = a / denomn
    iota_col = lax.broadcasted_iota(jnp.int32, (_B, 1), 0)
    oht = (brow == iota_col).astype(jnp.float32)
    rread = jnp.dot(oht, o * an, preferred_element_type=jnp.float32)
    res_ref[...] = (
        jnp.dot(jnp.broadcast_to(q, (_B, _DIM)), wq_ref[...],
                preferred_element_type=jnp.float32)
        + jnp.dot(rread, wr_ref[...], preferred_element_type=jnp.float32)
        + ob_ref[...])


def _full(shape):
    return pl.BlockSpec(shape, lambda *_: tuple(0 for _ in shape))


def kernel(x, edge_index, edge_attr, batch, lin0_W, lin0_b, net1_W, net1_b,
           net2_W, net2_b, root_W, conv_b, gru_Wih, gru_Whh, gru_bih, gru_bhh,
           lstm_Wih, lstm_Whh, lstm_bih, lstm_bhh, out_W, out_b):
    f32 = jnp.float32
    src = edge_index[0]
    dst = edge_index[1]
    srcp = jnp.pad(src, (0, _EP - _E)).reshape(_NIR, _IL)
    dstp = jnp.pad(dst, (0, _EP - _E), constant_values=_N).reshape(_NIR, _IL)
    eap = jnp.pad(edge_attr, ((0, _EP - _E), (0, 4)))
    xp = jnp.pad(x, ((0, _NP - _N), (0, 2)))

    lin0_Wt = jnp.pad(lin0_W.T, ((0, 2), (0, 0)))
    lin0_b2 = lin0_b.reshape(1, _DIM)
    net1_Wt = jnp.pad(net1_W.T, ((0, 4), (0, 0)))
    net1_b2 = net1_b.reshape(1, _DIM)
    w3t = net2_W.reshape(_DIM, _DIM, _DIM).transpose(2, 0, 1).reshape(
        _DIM * _DIM, _DIM).T
    b2t = net2_b.reshape(_DIM, _DIM).T
    root_Wt = root_W.T
    conv_b2 = conv_b.reshape(1, _DIM)
    wirt = gru_Wih[0:_DIM].T
    wizt = gru_Wih[_DIM:2 * _DIM].T
    wint = gru_Wih[2 * _DIM:].T
    whrt = gru_Whh[0:_DIM].T
    whzt = gru_Whh[_DIM:2 * _DIM].T
    whnt = gru_Whh[2 * _DIM:].T
    bir = gru_bih[0:_DIM].reshape(1, _DIM)
    biz = gru_bih[_DIM:2 * _DIM].reshape(1, _DIM)
    binn = gru_bih[2 * _DIM:].reshape(1, _DIM)
    bhr = gru_bhh[0:_DIM].reshape(1, _DIM)
    bhz = gru_bhh[_DIM:2 * _DIM].reshape(1, _DIM)
    bhn = gru_bhh[2 * _DIM:].reshape(1, _DIM)
    lstm_b = (lstm_bih + lstm_bhh)
    gi_b = lstm_b[0:_DIM].reshape(1, _DIM)
    gf_b = lstm_b[_DIM:2 * _DIM].reshape(1, _DIM)
    gg_b = lstm_b[2 * _DIM:3 * _DIM].reshape(1, _DIM)
    go_b = lstm_b[3 * _DIM:].reshape(1, _DIM)
    out_Wt = out_W.T
    wq = out_Wt[0:_DIM]
    wr = out_Wt[_DIM:]
    ob = out_b.reshape(1, 2)
    bcol = batch.reshape(_N, 1)
    brow = batch.reshape(1, _N)

    zeros_nd = jnp.zeros((_NP, _DW), f32)
    ones_e = jnp.ones((_EP, _DW), f32)

    srcA, srcB = srcp[:_NIR // 2], srcp[_NIR // 2:]
    dstA, dstB = dstp[:_NIR // 2], dstp[_NIR // 2:]

    _, sc_scatter_full = _sc_kernels(_NIR)
    sc_gather, sc_scatter = _sc_kernels(_NIR // 2)
    degp = sc_scatter_full(ones_e, dstp, zeros_nd)

    out0, invdeg = pl.pallas_call(
        _prep_body,
        in_specs=[_full((_NP, 16)), _full((16, _DIM)), _full((1, _DIM)),
                  _full((_NC, _NP, _DW))],
        out_specs=[_full((_NP, _DW)), _full((_NP, _DIM))],
        out_shape=[jax.ShapeDtypeStruct((_NP, _DW), f32),
                   jax.ShapeDtypeStruct((_NP, _DIM), f32)],
    )(xp, lin0_Wt, lin0_b2, degp)

    h1t = pl.pallas_call(
        _h1_body,
        grid=(_EP // _EB1,),
        in_specs=[pl.BlockSpec((_EB1, 8), lambda i: (i, 0)),
                  pl.BlockSpec((8, _DIM), lambda i: (0, 0)),
                  pl.BlockSpec((1, _DIM), lambda i: (0, 0))],
        out_specs=pl.BlockSpec((_DIM, _EB1), lambda i: (0, i)),
        out_shape=jax.ShapeDtypeStruct((_DIM, _EP), f32),
    )(eap, net1_Wt, net1_b2)

    def _msg_half(off):
        return pl.pallas_call(
            _msg_body,
            grid=(_EPH // _EB,),
            in_specs=[pl.BlockSpec((_DIM, _EB), lambda i: (0, i + off)),
                      pl.BlockSpec((_EB, _DW), lambda i: (i, 0)),
                      pl.BlockSpec((_DIM, _DIM * _DIM), lambda i: (0, 0)),
                      pl.BlockSpec((_DIM, _DIM), lambda i: (0, 0))],
            out_specs=pl.BlockSpec((_EB, _DW), lambda i: (i, 0)),
            out_shape=jax.ShapeDtypeStruct((_EPH, _DW), f32),
        )

    msg_a = _msg_half(0)
    msg_b = _msg_half(_EPH // _EB)

    _NRB = _NP // 8
    node_call = pl.pallas_call(
        _node_body,
        grid=(8,),
        in_specs=[pl.BlockSpec((_NC, _NRB, _DW), lambda i: (0, i, 0)),
                  pl.BlockSpec((_NC, _NRB, _DW), lambda i: (0, i, 0)),
                  pl.BlockSpec((_NRB, _DIM), lambda i: (i, 0)),
                  pl.BlockSpec((_NRB, _DW), lambda i: (i, 0)),
                  pl.BlockSpec((_NRB, _DW), lambda i: (i, 0))]
        + [pl.BlockSpec((_DIM, _DIM), lambda i: (0, 0))]
        + [pl.BlockSpec((1, _DIM), lambda i: (0, 0))]
        + [pl.BlockSpec((_DIM, _DIM), lambda i: (0, 0))] * 6
        + [pl.BlockSpec((1, _DIM), lambda i: (0, 0))] * 6,
        out_specs=pl.BlockSpec((_NRB, _DW), lambda i: (i, 0)),
        out_shape=jax.ShapeDtypeStruct((_NP, _DW), f32),
    )

    h = out0
    for _ in range(6):
        ga = sc_gather(h, srcA)
        ma = msg_a(h1t, ga, w3t, b2t)
        gb = sc_gather(h, srcB)
        pa = sc_scatter(ma, dstA, zeros_nd)
        mb = msg_b(h1t, gb, w3t, b2t)
        pb = sc_scatter(mb, dstB, zeros_nd)
        h = node_call(pa, pb, invdeg, h, h, root_Wt, conv_b2,
                      wirt, wizt, wint, whrt, whzt, whnt,
                      bir, biz, binn, bhr, bhz, bhn)

    res = pl.pallas_call(
        _s2s_body,
        in_specs=[_full((_NP, _DW)), _full((_N, 1)), _full((1, _N))]
        + [_full((1, _DIM))] * 4 + [_full((_DIM, 2))] * 2 + [_full((1, 2))],
        out_specs=_full((_B, 2)),
        out_shape=jax.ShapeDtypeStruct((_B, 2), f32),
    )(h, bcol, brow, gi_b, gf_b, gg_b, go_b, wq, wr, ob)
    return res


# speedup vs baseline: 4.3989x; 1.0046x over previous
import functools

import jax
import jax.numpy as jnp
from jax import lax
from jax.experimental import pallas as pl
from jax.experimental.pallas import tpu as pltpu
from jax.experimental.pallas import tpu_sc as plsc

_N = 10000
_E = 160000
_DIM = 32
_B = 64

_NC = 2
_NS = 16
_NW = _NC * _NS
_IL = 128
_EP = 163840
_NIR = _EP // _IL
_RPT = _NIR // _NW
_NP = 10112
_RPS = _NP // _NS

_DW = 128
_EPH = _EP // 2
_EB = 512
_EB1 = 2048

def _leaky(v):
    return jnp.where(v >= 0, v, 0.01 * v)


@functools.cache
def _sc_kernels(nir, chained=False):
    mesh = plsc.VectorSubcoreMesh(core_axis_name="c", subcore_axis_name="s",
                                  num_cores=_NC, num_subcores=_NS)
    rpt = nir // _NW
    nep = nir * _IL
    nld = rpt if rpt % 8 == 0 else rpt + 8 - rpt % 8

    def _idx_window(idx, idxbuf, wid):
        base = wid * rpt
        off = base % 8 if rpt % 8 else 0
        start = pl.multiple_of(base - off, 8)
        pltpu.sync_copy(idx.at[pl.ds(start, nld)], idxbuf)
        return off

    nbuf = 2
    ngrp = rpt // nbuf

    @functools.partial(
        pl.kernel,
        out_type=jax.ShapeDtypeStruct((nep, _DW), jnp.float32),
        mesh=mesh,
        scratch_types=[pltpu.VMEM((nld, _IL), jnp.int32)]
        + [pltpu.VMEM((_IL, _DW), jnp.float32)] * nbuf
        + [pltpu.SemaphoreType.DMA] * (2 * nbuf)
        + [pltpu.VMEM_SHARED((_NP, _DW), jnp.float32)],
    )
    def sc_gather(table, idx, out, idxbuf, *bufs_sems):
        rbs = bufs_sems[:nbuf]
        gss = bufs_sems[nbuf:2 * nbuf]
        sss = bufs_sems[2 * nbuf:2 * nbuf + nbuf]
        tab = bufs_sems[2 * nbuf + nbuf]
        sid = lax.axis_index("s")
        wid = sid * _NC + lax.axis_index("c")
        base = wid * rpt
        pltpu.sync_copy(table.at[pl.ds(sid * _RPS, _RPS)],
                        tab.at[pl.ds(sid * _RPS, _RPS)])
        off = _idx_window(idx, idxbuf, wid)
        plsc.subcore_barrier()
        for b in range(nbuf):
            pltpu.async_copy(tab.at[idxbuf.at[off + b]], rbs[b], gss[b])

        def outer(g, carry):
            for b in range(nbuf):
                j = g * nbuf + b
                pltpu.make_async_copy(tab.at[idxbuf.at[off + b]], rbs[b],
                                      gss[b]).wait()
                pltpu.async_copy(rbs[b], out.at[pl.ds((base + j) * _IL, _IL)],
                                 sss[b])

            @pl.when(g < ngrp - 1)
            def _():
                for b in range(nbuf):
                    j = g * nbuf + b
                    pltpu.make_async_copy(
                        rbs[b], out.at[pl.ds((base + j) * _IL, _IL)],
                        sss[b]).wait()
                    pltpu.async_copy(tab.at[idxbuf.at[off + (g + 1) * nbuf + b]],
                                     rbs[b], gss[b])

            return carry

        lax.fori_loop(0, ngrp, outer, 0)
        for b in range(nbuf):
            j = (ngrp - 1) * nbuf + b
            pltpu.make_async_copy(rbs[b], out.at[pl.ds((base + j) * _IL, _IL)],
                                  sss[b]).wait()

    nmb = 2
    ngrp2 = rpt // nmb
    tail = rpt - ngrp2 * nmb

    @functools.partial(
        pl.kernel,
        out_type=jax.ShapeDtypeStruct((_NC, _NP, _DW), jnp.float32),
        mesh=mesh,
        scratch_types=[pltpu.VMEM((nld, _IL), jnp.int32)]
        + [pltpu.VMEM((_IL, _DW), jnp.float32)] * nmb
        + [pltpu.SemaphoreType.DMA] * nmb
        + [pltpu.VMEM_SHARED((_NP, _DW), jnp.float32)],
    )
    def sc_scatter(msgs, idx, zeros, out, idxbuf, *bufs):
        mbs = bufs[:nmb]
        lss = bufs[nmb:2 * nmb]
        acc = bufs[2 * nmb]
        cid = lax.axis_index("c")
        sid = lax.axis_index("s")
        wid = sid * _NC + cid
        init_src = (zeros.at[cid] if chained else zeros)
        pltpu.sync_copy(init_src.at[pl.ds(sid * _RPS, _RPS)],
                        acc.at[pl.ds(sid * _RPS, _RPS)])
        plsc.subcore_barrier()
        base = wid * rpt
        off = _idx_window(idx, idxbuf, wid)
        for b in range(nmb):
            pltpu.async_copy(msgs.at[pl.ds((base + b) * _IL, _IL)],
                             mbs[b], lss[b])

        def step(c, b):
            pltpu.make_async_copy(msgs.at[pl.ds((base + c) * _IL, _IL)],
                                  mbs[b], lss[b]).wait()
            pltpu.sync_copy(mbs[b], acc.at[idxbuf.at[off + c]], add=True)

            @pl.when(c < rpt - nmb)
            def _():
                pltpu.async_copy(msgs.at[pl.ds((base + c + nmb) * _IL, _IL)],
                                 mbs[b], lss[b])

        def outer(g, carry):
            for b in range(nmb):
                step(g * nmb + b, b)
            return carry

        lax.fori_loop(0, ngrp2, outer, 0)
        for b in range(tail):
            step(ngrp2 * nmb + b, b)
        plsc.subcore_barrier()
        pltpu.sync_copy(acc.at[pl.ds(sid * _RPS, _RPS)],
                        out.at[cid].at[pl.ds(sid * _RPS, _RPS)])

    return sc_gather, sc_scatter


def _pad_dw(v):
    return jnp.concatenate(
        [v, jnp.zeros((v.shape[0], _DW - _DIM), jnp.float32)], axis=1)


def _prep_body(x_ref, w_ref, b_ref, dp_ref, out_ref, inv_ref):
    o = _leaky(
        jnp.dot(x_ref[...], w_ref[...], preferred_element_type=jnp.float32)
        + b_ref[...])
    out_ref[...] = _pad_dw(o)
    deg = dp_ref[0, :, 0:_DIM] + dp_ref[1, :, 0:_DIM]
    inv_ref[...] = 1.0 / jnp.maximum(deg, 1.0)


def _h1_body(ea_ref, w_ref, b_ref, h1_ref):
    h1 = _leaky(
        jnp.dot(ea_ref[...], w_ref[...], preferred_element_type=jnp.float32)
        + b_ref[...])
    h1_ref[...] = h1.T


def _msg_body(h1t_ref, g_ref, w3t_ref, b2t_ref, msg_ref):
    h1t = h1t_ref[...]
    gt = g_ref[:, 0:_DIM].T
    pt = (jnp.reshape(jnp.broadcast_to(h1t[:, None, :], (_DIM, _DIM, _EB)),
                      (_DIM * _DIM, _EB))
          * jnp.reshape(jnp.broadcast_to(gt[None, :, :], (_DIM, _DIM, _EB)),
                        (_DIM * _DIM, _EB)))
    msgt = (jnp.dot(w3t_ref[...], pt, preferred_element_type=jnp.float32)
            + jnp.dot(b2t_ref[...], gt, preferred_element_type=jnp.float32))
    msg_ref[...] = _pad_dw(msgt.T)


def _node_body(dp, inv, out, h, rwt, cb, wirt, wizt, wint, whrt, whzt,
               whnt, bir, biz, binn, bhr, bhz, bhn, h_new):
    agg = (dp[0, :, 0:_DIM] + dp[1, :, 0:_DIM]) * inv[...]
    o = out[:, 0:_DIM]
    hh = h[:, 0:_DIM]
    m = _leaky(agg + jnp.dot(o, rwt[...], preferred_element_type=jnp.float32)
               + cb[...])
    gh_r = jnp.dot(hh, whrt[...], preferred_element_type=jnp.float32) + bhr[...]
    gh_z = jnp.dot(hh, whzt[...], preferred_element_type=jnp.float32) + bhz[...]
    gh_n = jnp.dot(hh, whnt[...], preferred_element_type=jnp.float32) + bhn[...]
    r = jax.nn.sigmoid(
        jnp.dot(m, wirt[...], preferred_element_type=jnp.float32) + bir[...]
        + gh_r)
    z = jax.nn.sigmoid(
        jnp.dot(m, wizt[...], preferred_element_type=jnp.float32) + biz[...]
        + gh_z)
    n = jnp.tanh(
        jnp.dot(m, wint[...], preferred_element_type=jnp.float32) + binn[...]
        + r * gh_n)
    h_new[...] = _pad_dw((1.0 - z) * n + z * hh)


def _s2s_body(out_ref, bcol_ref, brow_ref, gi_ref, gf_ref, gg_ref, go_ref,
              wq_ref, wr_ref, ob_ref, res_ref):
    o = out_ref[0:_N, 0:_DIM]
    bcol = bcol_ref[...]
    brow = brow_ref[...]
    i_g = jax.nn.sigmoid(gi_ref[...])
    f_g = jax.nn.sigmoid(gf_ref[...])
    g_g = jnp.tanh(gg_ref[...])
    o_g = jax.nn.sigmoid(go_ref[...])
    del f_g
    q = o_g * jnp.tanh(i_g * g_g)
    e = jnp.sum(o * q, axis=1, keepdims=True)
    iota_row = lax.broadcasted_iota(jnp.int32, (1, _B), 1)
    ohb = bcol == iota_row
    ohf = ohb.astype(jnp.float32)
    neg = jnp.float32(-jnp.inf)
    emax = jnp.max(jnp.where(ohb, e, neg), axis=0, keepdims=True)
    emax = jnp.where(jnp.isfinite(emax), emax, 0.0)
    emaxn = jnp.sum(ohf * emax, axis=1, keepdims=True)
    a = jnp.exp(e - emaxn)
    denom = jnp.sum(ohf * a, axis=0, keepdims=True)
    denomn = jnp.sum(ohf * jnp.maximum(denom, 1e-16), axis=1, keepdims=True)
    an = a / denomn
    iota_col = lax.broadcasted_iota(jnp.int32, (_B, 1), 0)
    oht = (brow == iota_col).astype(jnp.float32)
    rread = jnp.dot(oht, o * an, preferred_element_type=jnp.float32)
    res_ref[...] = (
        jnp.dot(jnp.broadcast_to(q, (_B, _DIM)), wq_ref[...],
                preferred_element_type=jnp.float32)
        + jnp.dot(rread, wr_ref[...], preferred_element_type=jnp.float32)
        + ob_ref[...])


def _full(shape):
    return pl.BlockSpec(shape, lambda *_: tuple(0 for _ in shape))


def kernel(x, edge_index, edge_attr, batch, lin0_W, lin0_b, net1_W, net1_b,
           net2_W, net2_b, root_W, conv_b, gru_Wih, gru_Whh, gru_bih, gru_bhh,
           lstm_Wih, lstm_Whh, lstm_bih, lstm_bhh, out_W, out_b):
    f32 = jnp.float32
    src = edge_index[0]
    dst = edge_index[1]
    srcp = jnp.pad(src, (0, _EP - _E)).reshape(_NIR, _IL)
    dstp = jnp.pad(dst, (0, _EP - _E), constant_values=_N).reshape(_NIR, _IL)
    eap = jnp.pad(edge_attr, ((0, _EP - _E), (0, 4)))
    xp = jnp.pad(x, ((0, _NP - _N), (0, 2)))

    lin0_Wt = jnp.pad(lin0_W.T, ((0, 2), (0, 0)))
    lin0_b2 = lin0_b.reshape(1, _DIM)
    net1_Wt = jnp.pad(net1_W.T, ((0, 4), (0, 0)))
    net1_b2 = net1_b.reshape(1, _DIM)
    w3t = net2_W.reshape(_DIM, _DIM, _DIM).transpose(2, 0, 1).reshape(
        _DIM * _DIM, _DIM).T
    b2t = net2_b.reshape(_DIM, _DIM).T
    root_Wt = root_W.T
    conv_b2 = conv_b.reshape(1, _DIM)
    wirt = gru_Wih[0:_DIM].T
    wizt = gru_Wih[_DIM:2 * _DIM].T
    wint = gru_Wih[2 * _DIM:].T
    whrt = gru_Whh[0:_DIM].T
    whzt = gru_Whh[_DIM:2 * _DIM].T
    whnt = gru_Whh[2 * _DIM:].T
    bir = gru_bih[0:_DIM].reshape(1, _DIM)
    biz = gru_bih[_DIM:2 * _DIM].reshape(1, _DIM)
    binn = gru_bih[2 * _DIM:].reshape(1, _DIM)
    bhr = gru_bhh[0:_DIM].reshape(1, _DIM)
    bhz = gru_bhh[_DIM:2 * _DIM].reshape(1, _DIM)
    bhn = gru_bhh[2 * _DIM:].reshape(1, _DIM)
    lstm_b = (lstm_bih + lstm_bhh)
    gi_b = lstm_b[0:_DIM].reshape(1, _DIM)
    gf_b = lstm_b[_DIM:2 * _DIM].reshape(1, _DIM)
    gg_b = lstm_b[2 * _DIM:3 * _DIM].reshape(1, _DIM)
    go_b = lstm_b[3 * _DIM:].reshape(1, _DIM)
    out_Wt = out_W.T
    wq = out_Wt[0:_DIM]
    wr = out_Wt[_DIM:]
    ob = out_b.reshape(1, 2)
    bcol = batch.reshape(_N, 1)
    brow = batch.reshape(1, _N)

    zeros_nd = jnp.zeros((_NP, _DW), f32)
    ones_e = jnp.ones((_EP, _DW), f32)

    srcA, srcB = srcp[:_NIR // 2], srcp[_NIR // 2:]
    dstA, dstB = dstp[:_NIR // 2], dstp[_NIR // 2:]

    _, sc_scatter_full = _sc_kernels(_NIR)
    sc_gather, sc_scatter = _sc_kernels(_NIR // 2)
    _, sc_scatter_chained = _sc_kernels(_NIR // 2, chained=True)
    degp = sc_scatter_full(ones_e, dstp, zeros_nd)

    out0, invdeg = pl.pallas_call(
        _prep_body,
        in_specs=[_full((_NP, 16)), _full((16, _DIM)), _full((1, _DIM)),
                  _full((_NC, _NP, _DW))],
        out_specs=[_full((_NP, _DW)), _full((_NP, _DIM))],
        out_shape=[jax.ShapeDtypeStruct((_NP, _DW), f32),
                   jax.ShapeDtypeStruct((_NP, _DIM), f32)],
    )(xp, lin0_Wt, lin0_b2, degp)

    h1t = pl.pallas_call(
        _h1_body,
        grid=(_EP // _EB1,),
        in_specs=[pl.BlockSpec((_EB1, 8), lambda i: (i, 0)),
                  pl.BlockSpec((8, _DIM), lambda i: (0, 0)),
                  pl.BlockSpec((1, _DIM), lambda i: (0, 0))],
        out_specs=pl.BlockSpec((_DIM, _EB1), lambda i: (0, i)),
        out_shape=jax.ShapeDtypeStruct((_DIM, _EP), f32),
    )(eap, net1_Wt, net1_b2)

    def _msg_half(off):
        return pl.pallas_call(
            _msg_body,
            grid=(_EPH // _EB,),
            in_specs=[pl.BlockSpec((_DIM, _EB), lambda i: (0, i + off)),
                      pl.BlockSpec((_EB, _DW), lambda i: (i, 0)),
                      pl.BlockSpec((_DIM, _DIM * _DIM), lambda i: (0, 0)),
                      pl.BlockSpec((_DIM, _DIM), lambda i: (0, 0))],
            out_specs=pl.BlockSpec((_EB, _DW), lambda i: (i, 0)),
            out_shape=jax.ShapeDtypeStruct((_EPH, _DW), f32),
        )

    msg_a = _msg_half(0)
    msg_b = _msg_half(_EPH // _EB)

    _NRB = _NP // 8
    node_call = pl.pallas_call(
        _node_body,
        grid=(8,),
        in_specs=[pl.BlockSpec((_NC, _NRB, _DW), lambda i: (0, i, 0)),
                  pl.BlockSpec((_NRB, _DIM), lambda i: (i, 0)),
                  pl.BlockSpec((_NRB, _DW), lambda i: (i, 0)),
                  pl.BlockSpec((_NRB, _DW), lambda i: (i, 0))]
        + [pl.BlockSpec((_DIM, _DIM), lambda i: (0, 0))]
        + [pl.BlockSpec((1, _DIM), lambda i: (0, 0))]
        + [pl.BlockSpec((_DIM, _DIM), lambda i: (0, 0))] * 6
        + [pl.BlockSpec((1, _DIM), lambda i: (0, 0))] * 6,
        out_specs=pl.BlockSpec((_NRB, _DW), lambda i: (i, 0)),
        out_shape=jax.ShapeDtypeStruct((_NP, _DW), f32),
    )

    h = out0
    for _ in range(6):
        ga = sc_gather(h, srcA)
        ma = msg_a(h1t, ga, w3t, b2t)
        gb = sc_gather(h, srcB)
        pa = sc_scatter(ma, dstA, zeros_nd)
        mb = msg_b(h1t, gb, w3t, b2t)
        pb = sc_scatter_chained(mb, dstB, pa)
        h = node_call(pb, invdeg, h, h, root_Wt, conv_b2,
                      wirt, wizt, wint, whrt, whzt, whnt,
                      bir, biz, binn, bhr, bhz, bhn)

    res = pl.pallas_call(
        _s2s_body,
        in_specs=[_full((_NP, _DW)), _full((_N, 1)), _full((1, _N))]
        + [_full((1, _DIM))] * 4 + [_full((_DIM, 2))] * 2 + [_full((1, 2))],
        out_specs=_full((_B, 2)),
        out_shape=jax.ShapeDtypeStruct((_B, 2), f32),
    )(h, bcol, brow, gi_b, gf_b, gg_b, go_b, wq, wr, ob)
    return res
